# Initial kernel scaffold; baseline (speedup 1.0000x reference)
import jax
import jax.numpy as jnp
from jax.experimental import pallas as pl


def kernel(Sij, Cijj, dst, Wc, alphaC, alphaf, b, w, Wf, bias, W1, b1, W2, b2):
    raise NotImplementedError("write your pallas kernel here")


# speedup vs baseline: 22.0900x; 22.0900x over previous
import functools

import jax
import jax.numpy as jnp
from jax import lax
from jax.experimental import pallas as pl
from jax.experimental.pallas import tpu as pltpu
from jax.experimental.pallas import tpu_sc as plsc

N = 10000
E = 160000
D = 16
DD = 256
HALF = 128
NSUB = 16
EPT = E // NSUB
K = 125
NCHUNK = EPT // K
RPT = N // NSUB
RB = 125


def _sc_corr_kernel(s_hbm, c_hbm, dst_hbm, out_hbm, idx_v, sbuf, cbuf, acc):
    cid = lax.axis_index("c")
    sid = lax.axis_index("s")
    col0 = cid * HALF
    ebase = sid * EPT
    rbase = sid * RPT

    pltpu.sync_copy(dst_hbm.at[pl.ds(sid * NCHUNK, NCHUNK)], idx_v)

    zero = jnp.zeros((16,), jnp.float32)

    def _zrow(i, _):
        for j in range(HALF // 16):
            sbuf[i, pl.ds(j * 16, 16)] = zero
        return 0

    lax.fori_loop(0, RB, _zrow, 0)
    for r in range(RPT // RB):
        pltpu.sync_copy(sbuf, acc.at[pl.ds(rbase + r * RB, RB)])
    plsc.subcore_barrier()

    def _chunk(t, _):
        e0 = ebase + t * K
        pltpu.sync_copy(s_hbm.at[pl.ds(e0, K), pl.ds(col0, HALF)], sbuf)
        pltpu.sync_copy(c_hbm.at[pl.ds(e0, K), pl.ds(col0, HALF)], cbuf)

        def _mrow(i, _):
            for j in range(HALF // 16):
                sl = pl.ds(j * 16, 16)
                sbuf[i, sl] = sbuf[i, sl] * cbuf[i, sl]
            return 0

        lax.fori_loop(0, K, _mrow, 0)
        pltpu.sync_copy(sbuf, acc.at[idx_v.at[t]], add=True)
        return 0

    lax.fori_loop(0, NCHUNK, _chunk, 0)
    plsc.subcore_barrier()

    pltpu.sync_copy(acc.at[pl.ds(rbase, RPT)],
                    out_hbm.at[pl.ds(rbase, RPT), pl.ds(col0, HALF)])


@functools.cache
def _sc_corr():
    return functools.partial(
        pl.kernel,
        mesh=plsc.VectorSubcoreMesh(core_axis_name="c", subcore_axis_name="s"),
        out_type=jax.ShapeDtypeStruct((N, DD), jnp.float32),
        scratch_types=[
            pltpu.VMEM((NCHUNK, K), jnp.int32),
            pltpu.VMEM((K, HALF), jnp.float32),
            pltpu.VMEM((K, HALF), jnp.float32),
            pltpu.VMEM_SHARED((N, HALF), jnp.float32),
        ],
        compiler_params=pltpu.CompilerParams(use_tc_tiling_on_sc=False),
    )(_sc_corr_kernel)


def _tc_tail_kernel(corr_ref, m_ref, bd1_ref, b1_ref, bd2_ref, b2_ref,
                    wft_ref, biast_ref, out_ref):
    corr = corr_ref[...]
    scores = lax.dot_general(corr, m_ref[...],
                             (((1,), (0,)), ((), ())),
                             preferred_element_type=jnp.float32)
    s = scores - jnp.max(scores)
    e = jnp.exp(s)
    att = e / jnp.sum(e)
    ci = lax.dot_general(att, corr, (((0,), (0,)), ((), ())),
                         preferred_element_type=jnp.float32)
    fi = lax.dot_general(ci, bd1_ref[...], (((1,), (0,)), ((), ())),
                         preferred_element_type=jnp.float32)
    fi = jnp.maximum(fi + b1_ref[...], 0.0)
    fi = 1.0 / (1.0 + jnp.exp(-fi))
    fi2 = lax.dot_general(fi, bd2_ref[...], (((1,), (0,)), ((), ())),
                          preferred_element_type=jnp.float32)
    fi2 = jnp.maximum(fi2 + b2_ref[...], 0.0)
    out = lax.dot_general(fi2, wft_ref[...], (((1,), (0,)), ((), ())),
                          preferred_element_type=jnp.float32)
    out_ref[...] = out + biast_ref[...]


def kernel(Sij, Cijj, dst, Wc, alphaC, alphaf, b, w, Wf, bias, W1, b1, W2, b2):
    S2 = Sij.reshape(E, DD)
    C2 = Cijj.reshape(E, DD)
    dst2 = dst.reshape(E // K, K)

    corr = _sc_corr()(S2, C2, dst2)

    u = (w @ Wc).reshape(D)
    v = alphaC.reshape(D)
    m = (u[:, None] * v[None, :]).reshape(DD, 1)
    eye_d = jnp.eye(D, dtype=jnp.float32)
    bd1 = (eye_d[:, None, :, None] * W1.T[None, :, None, :]).reshape(DD, D * 256)
    b1big = jnp.tile(b1, D).reshape(1, D * 256)
    eye_f = jnp.eye(256, dtype=jnp.float32)
    bd2 = (W2.reshape(D, 1, 1) * eye_f[None, :, :]).reshape(D * 256, 256)
    b2big = jnp.full((1, 256), b2[0], dtype=jnp.float32)

    P = pl.pallas_call(
        _tc_tail_kernel,
        out_shape=jax.ShapeDtypeStruct((1, D), jnp.float32),
    )(corr, m, bd1, b1big, bd2, b2big, Wf.T, bias.T)
    return P
